# baseline (speedup 1.0000x reference)
import functools

import jax
import jax.numpy as jnp
from jax import lax
from jax.experimental import pallas as pl
from jax.experimental.pallas import tpu as pltpu
from jax.experimental.pallas import tpu_sc as plsc

NUM_NODES = 1000000
EMBED_DIM = 32
NUM_ENT = 16384

NC, NS = 2, 16
NW = NC * NS
B_PER_W = NUM_ENT // NW
N_TC = 7813
TC_BASE = N_TC // NW
TC_REM = N_TC - TC_BASE * NW
NG = 62
HCAP = 1152
OCAP = 128
IMG_PAD = 2048
IMG_H = NUM_ENT + IMG_PAD

_mesh = plsc.VectorSubcoreMesh(core_axis_name="c", subcore_axis_name="s")


def _wid_lo():
    wid = lax.axis_index("s") * NC + lax.axis_index("c")
    lo = wid * TC_BASE + jnp.minimum(wid, TC_REM)
    hi = lo + TC_BASE + jnp.where(wid < TC_REM, 1, 0)
    return wid, lo, hi


@functools.partial(
    pl.kernel,
    mesh=_mesh,
    out_type=(
        jax.ShapeDtypeStruct((NW * HCAP,), jnp.int32),
        jax.ShapeDtypeStruct((NW * OCAP,), jnp.int32),
    ),
    scratch_types=[
        pltpu.VMEM((NUM_ENT,), jnp.int32),
        pltpu.VMEM((HCAP,), jnp.int32),
        pltpu.VMEM((HCAP,), jnp.int32),
        pltpu.VMEM((OCAP,), jnp.int32),
    ],
    compiler_params=pltpu.CompilerParams(
        use_tc_tiling_on_sc=False, needs_layout_passes=False),
)
def _k0(nodes_hbm, hko_hbm, offs_hbm, nv, hk1, hk2, offs_v):
    wid, lo, hi = _wid_lo()
    pltpu.sync_copy(nodes_hbm, nv)
    iota16 = lax.iota(jnp.int32, 16)

    def scan_body(i, off):
        c = nv[pl.ds(i * 16, 16)]
        tc = lax.shift_right_logical(c, 7)
        m = (tc >= lo) & (tc < hi)
        j = iota16 + i * 16
        key = (lax.shift_left(tc - lo, 21)
               | lax.shift_left(j, 7) | (c & 127))
        offc = jnp.minimum(off, HCAP - 16)
        plsc.store_compressed(hk1.at[pl.ds(offc, 16)], key, mask=m)
        return off + jnp.sum(jnp.where(m, 1, 0))

    nh = lax.fori_loop(0, NUM_ENT // 16, scan_body, jnp.int32(0), unroll=4)
    nsteps = lax.shift_right_logical(nh + 15, 4)

    def group_body(g, off2):
        gv = jnp.full((16,), 0, jnp.int32) + g
        ov = jnp.full((16,), 0, jnp.int32) + off2
        plsc.store_scatter(offs_v, [gv], ov, mask=iota16 < 1)

        def inner(i, o2):
            pos = i * 16 + iota16
            key = hk1[pl.ds(i * 16, 16)]
            tcl = lax.shift_right_logical(key, 21)
            m = (pos < nh) & (lax.shift_right_logical(tcl, 2) == g)
            oc = jnp.minimum(o2, HCAP - 16)
            plsc.store_compressed(hk2.at[pl.ds(oc, 16)], key, mask=m)
            return o2 + jnp.sum(jnp.where(m, 1, 0))

        return lax.fori_loop(0, nsteps, inner, off2)

    off_end = lax.fori_loop(0, NG, group_body, jnp.int32(0))
    gv = jnp.full((16,), 0, jnp.int32) + NG
    ov = jnp.full((16,), 0, jnp.int32) + off_end
    plsc.store_scatter(offs_v, [gv], ov, mask=iota16 < 1)

    pltpu.sync_copy(hk2, hko_hbm.at[pl.ds(wid * HCAP, HCAP)])
    pltpu.sync_copy(offs_v, offs_hbm.at[pl.ds(wid * OCAP, OCAP)])


@functools.partial(
    pl.kernel,
    mesh=_mesh,
    out_type=jax.ShapeDtypeStruct((IMG_H, 128), jnp.float32),
    scratch_types=[
        pltpu.VMEM((HCAP,), jnp.int32),
        pltpu.VMEM((OCAP,), jnp.int32),
        [[[pltpu.VMEM((8, 128), jnp.float32) for _ in range(4)]
          for _ in range(4)] for _ in range(2)],
        [[[pltpu.VMEM((8, 128), jnp.float32) for _ in range(4)]
          for _ in range(4)] for _ in range(2)],
        [[pltpu.VMEM((8, 128), jnp.float32) for _ in range(2)]
         for _ in range(4)],
        [[pltpu.VMEM((8,), jnp.int32) for _ in range(2)] for _ in range(4)],
        [pltpu.SemaphoreType.DMA for _ in range(2)],
        [pltpu.SemaphoreType.DMA for _ in range(4)],
    ],
    compiler_params=pltpu.CompilerParams(
        needs_layout_passes=False, disable_bounds_checks=True),
)
def _k1(hko_hbm, offs_hbm, ft_hbm, pt_hbm, img_hbm,
        hkv, offs_v, fb, pb, stg, jbufs, sems, ssems):
    wid, lo, hi = _wid_lo()
    pltpu.sync_copy(hko_hbm.at[pl.ds(wid * HCAP, HCAP)], hkv)
    pltpu.sync_copy(offs_hbm.at[pl.ds(wid * OCAP, OCAP)], offs_v)
    iota16 = lax.iota(jnp.int32, 16)
    lane_lo = iota16 < 8
    lane_hi = iota16 >= 8

    def fire_set(g, s):
        for q in range(4):
            tc = jnp.minimum(lo + g * 4 + q, N_TC - 1)
            cstart = pl.multiple_of(tc * 128, 128)
            for tr in range(4):
                pltpu.make_async_copy(
                    ft_hbm.at[pl.ds(tr * 8, 8), pl.ds(cstart, 128)],
                    fb[s][tr][q], sems[s]).start()
                pltpu.make_async_copy(
                    pt_hbm.at[pl.ds(tr * 8, 8), pl.ds(cstart, 128)],
                    pb[s][tr][q], sems[s]).start()

    def wait_set(s):
        for q in range(4):
            for tr in range(4):
                pltpu.make_async_copy(
                    ft_hbm.at[pl.ds(0, 8), pl.ds(0, 128)],
                    fb[s][tr][q], sems[s]).wait()
                pltpu.make_async_copy(
                    pt_hbm.at[pl.ds(0, 8), pl.ds(0, 128)],
                    pb[s][tr][q], sems[s]).wait()

    def extract(g, s):
        gi = jnp.full((16,), 0, jnp.int32) + g + jnp.minimum(iota16, 1)
        ov = plsc.load_gather(offs_v, [gi])
        os0 = ov[0]
        os1 = ov[1]
        nb = lax.shift_right_logical(os1 - os0 + 15, 4)

        def one_batch(b, z):
            for h in range(2):
                pltpu.make_async_copy(
                    stg[z][h], img_hbm.at[jbufs[z][h]], ssems[z]).wait()
            idx = os0 + b * 16 + iota16
            valid = idx < os1
            idxc = jnp.minimum(idx, jnp.maximum(os1 - 1, 0))
            kvec = plsc.load_gather(hkv, [idxc])
            tcq = lax.shift_right_logical(kvec, 21) - g * 4
            wcv = kvec & 127
            dmy = NUM_ENT + (((wid * NG + g + b) * 16 + iota16)
                             & (IMG_PAD - 1))
            jv = jnp.where(valid,
                           lax.shift_right_logical(kvec, 7) & 16383,
                           dmy)
            plsc.store_scatter(jbufs[z][0], [iota16], jv, mask=lane_lo)
            plsc.store_scatter(jbufs[z][1], [iota16 - 8], jv, mask=lane_hi)

            for d in range(EMBED_DIM):
                tr = d // 8
                rowv = jnp.full((16,), 0, jnp.int32) + (d % 8)
                gf = None
                gp = None
                for q in range(4):
                    fq = plsc.load_gather(fb[s][tr][q], [rowv, wcv])
                    pq = plsc.load_gather(pb[s][tr][q], [rowv, wcv])
                    if gf is None:
                        gf, gp = fq, pq
                    else:
                        sel = tcq == q
                        gf = jnp.where(sel, fq, gf)
                        gp = jnp.where(sel, pq, gp)
                sv = gf + gp
                dv = jnp.full((16,), 0, jnp.int32) + d
                plsc.store_scatter(stg[z][0], [iota16, dv], sv, mask=lane_lo)
                plsc.store_scatter(stg[z][1], [iota16 - 8, dv], sv,
                                   mask=lane_hi)

            for h in range(2):
                pltpu.make_async_copy(
                    stg[z][h], img_hbm.at[jbufs[z][h]], ssems[z]).start()

        def batch_quad(b4, carry2):
            for z in range(4):
                one_batch(b4 * 4 + z, z)
            return carry2

        lax.fori_loop(0, lax.shift_right_logical(nb + 3, 2), batch_quad, 0)

    dmy0 = NUM_ENT + ((wid * 64 + iota16) & (IMG_PAD - 1))
    for z in range(4):
        plsc.store_scatter(jbufs[z][0], [iota16], dmy0 + z * 16,
                           mask=lane_lo)
        plsc.store_scatter(jbufs[z][1], [iota16 - 8], dmy0 + z * 16 + 8,
                           mask=lane_hi)
        for h in range(2):
            pltpu.make_async_copy(
                stg[z][h], img_hbm.at[jbufs[z][h]], ssems[z]).start()

    fire_set(jnp.int32(0), 0)

    def pair_body(g2, carry):
        g0 = g2 * 2
        fire_set(g0 + 1, 1)
        wait_set(0)
        extract(g0, 0)
        fire_set(g0 + 2, 0)
        wait_set(1)
        extract(g0 + 1, 1)
        return carry

    lax.fori_loop(0, NG // 2, pair_body, 0)
    wait_set(0)
    for z in range(4):
        for h in range(2):
            pltpu.make_async_copy(
                stg[z][h], img_hbm.at[jbufs[z][h]], ssems[z]).wait()


@functools.partial(
    pl.kernel,
    mesh=_mesh,
    out_type=jax.ShapeDtypeStruct((EMBED_DIM, NUM_ENT), jnp.float32),
    scratch_types=[
        pltpu.VMEM((B_PER_W, EMBED_DIM), jnp.float32),
        pltpu.VMEM((EMBED_DIM, B_PER_W), jnp.float32),
    ],
    compiler_params=pltpu.CompilerParams(
        use_tc_tiling_on_sc=False, needs_layout_passes=False),
)
def _k2(img_hbm, out_hbm, buf, out_t):
    wid, _, _ = _wid_lo()
    base = wid * B_PER_W
    pltpu.sync_copy(
        img_hbm.at[pl.ds(base, B_PER_W), pl.ds(0, EMBED_DIM)], buf)
    iota16 = lax.iota(jnp.int32, 16)

    def body(i, carry):
        d = i & 31
        e16 = i >> 5
        rowv = iota16 + e16 * 16
        colv = jnp.full((16,), 0, jnp.int32) + d
        out_t[d, pl.ds(e16 * 16, 16)] = plsc.load_gather(buf, [rowv, colv])
        return carry

    lax.fori_loop(0, EMBED_DIM * (B_PER_W // 16), body, 0, unroll=4)
    pltpu.sync_copy(out_t, out_hbm.at[:, pl.ds(base, B_PER_W)])


def kernel(nodes, feat_table, pos_table):
    n = nodes.astype(jnp.int32)
    hko, offs = _k0(n)
    img = _k1(hko, offs, feat_table.T, pos_table.T)
    return _k2(img)


# speedup vs baseline: 1.5606x; 1.5606x over previous
import functools

import jax
import jax.numpy as jnp
from jax import lax
from jax.experimental import pallas as pl
from jax.experimental.pallas import tpu as pltpu
from jax.experimental.pallas import tpu_sc as plsc

NUM_NODES = 1000000
EMBED_DIM = 32
NUM_ENT = 16384

NC, NS = 2, 16
NW = NC * NS
B_PER_W = NUM_ENT // NW
N_TC = 7813
TC_BASE = N_TC // NW
TC_REM = N_TC - TC_BASE * NW
NG = 62
HCAP = 1152
OCAP = 128
IMG_PAD = 2048
IMG_H = NUM_ENT + IMG_PAD

_mesh = plsc.VectorSubcoreMesh(core_axis_name="c", subcore_axis_name="s")


def _wid_lo():
    wid = lax.axis_index("s") * NC + lax.axis_index("c")
    lo = wid * TC_BASE + jnp.minimum(wid, TC_REM)
    hi = lo + TC_BASE + jnp.where(wid < TC_REM, 1, 0)
    return wid, lo, hi


@functools.partial(
    pl.kernel,
    mesh=_mesh,
    out_type=(
        jax.ShapeDtypeStruct((NW * HCAP,), jnp.int32),
        jax.ShapeDtypeStruct((NW * OCAP,), jnp.int32),
    ),
    scratch_types=[
        pltpu.VMEM((NUM_ENT,), jnp.int32),
        pltpu.VMEM((HCAP,), jnp.int32),
        pltpu.VMEM((HCAP,), jnp.int32),
        pltpu.VMEM((OCAP,), jnp.int32),
    ],
    compiler_params=pltpu.CompilerParams(
        use_tc_tiling_on_sc=False, needs_layout_passes=False),
)
def _k0(nodes_hbm, hko_hbm, offs_hbm, nv, hk1, hk2, offs_v):
    wid, lo, hi = _wid_lo()
    pltpu.sync_copy(nodes_hbm, nv)
    iota16 = lax.iota(jnp.int32, 16)

    def scan_body(i, off):
        c = nv[pl.ds(i * 16, 16)]
        tc = lax.shift_right_logical(c, 7)
        m = (tc >= lo) & (tc < hi)
        j = iota16 + i * 16
        key = (lax.shift_left(tc - lo, 21)
               | lax.shift_left(j, 7) | (c & 127))
        offc = jnp.minimum(off, HCAP - 16)
        plsc.store_compressed(hk1.at[pl.ds(offc, 16)], key, mask=m)
        return off + jnp.sum(jnp.where(m, 1, 0))

    nh = lax.fori_loop(0, NUM_ENT // 16, scan_body, jnp.int32(0), unroll=4)
    nsteps = lax.shift_right_logical(nh + 15, 4)

    def group_body(g, off2):
        gv = jnp.full((16,), 0, jnp.int32) + g
        ov = jnp.full((16,), 0, jnp.int32) + off2
        plsc.store_scatter(offs_v, [gv], ov, mask=iota16 < 1)

        def inner(i, o2):
            pos = i * 16 + iota16
            key = hk1[pl.ds(i * 16, 16)]
            tcl = lax.shift_right_logical(key, 21)
            m = (pos < nh) & (lax.shift_right_logical(tcl, 2) == g)
            oc = jnp.minimum(o2, HCAP - 16)
            plsc.store_compressed(hk2.at[pl.ds(oc, 16)], key, mask=m)
            return o2 + jnp.sum(jnp.where(m, 1, 0))

        return lax.fori_loop(0, nsteps, inner, off2)

    off_end = lax.fori_loop(0, NG, group_body, jnp.int32(0))
    gv = jnp.full((16,), 0, jnp.int32) + NG
    ov = jnp.full((16,), 0, jnp.int32) + off_end
    plsc.store_scatter(offs_v, [gv], ov, mask=iota16 < 1)

    pltpu.sync_copy(hk2, hko_hbm.at[pl.ds(wid * HCAP, HCAP)])
    pltpu.sync_copy(offs_v, offs_hbm.at[pl.ds(wid * OCAP, OCAP)])


@functools.partial(
    pl.kernel,
    mesh=_mesh,
    out_type=jax.ShapeDtypeStruct((IMG_H, 128), jnp.float32),
    scratch_types=[
        pltpu.VMEM((HCAP,), jnp.int32),
        pltpu.VMEM((OCAP,), jnp.int32),
        [[[pltpu.VMEM((8, 128), jnp.float32) for _ in range(4)]
          for _ in range(4)] for _ in range(2)],
        [[[pltpu.VMEM((8, 128), jnp.float32) for _ in range(4)]
          for _ in range(4)] for _ in range(2)],
        [[pltpu.VMEM((8, 128), jnp.float32) for _ in range(2)]
         for _ in range(2)],
        [[pltpu.VMEM((8,), jnp.int32) for _ in range(2)] for _ in range(2)],
        [pltpu.SemaphoreType.DMA for _ in range(2)],
        [pltpu.SemaphoreType.DMA for _ in range(2)],
    ],
    compiler_params=pltpu.CompilerParams(
        needs_layout_passes=False, disable_bounds_checks=True),
)
def _k1(hko_hbm, offs_hbm, ft_hbm, pt_hbm, img_hbm,
        hkv, offs_v, fb, pb, stg, jbufs, sems, ssems):
    wid, lo, hi = _wid_lo()
    pltpu.sync_copy(hko_hbm.at[pl.ds(wid * HCAP, HCAP)], hkv)
    pltpu.sync_copy(offs_hbm.at[pl.ds(wid * OCAP, OCAP)], offs_v)
    iota16 = lax.iota(jnp.int32, 16)
    lane_lo = iota16 < 8
    lane_hi = iota16 >= 8

    def fire_set(g, s):
        for q in range(4):
            tc = jnp.minimum(lo + g * 4 + q, N_TC - 1)
            cstart = pl.multiple_of(tc * 128, 128)
            for tr in range(4):
                pltpu.make_async_copy(
                    ft_hbm.at[pl.ds(tr * 8, 8), pl.ds(cstart, 128)],
                    fb[s][tr][q], sems[s]).start()
                pltpu.make_async_copy(
                    pt_hbm.at[pl.ds(tr * 8, 8), pl.ds(cstart, 128)],
                    pb[s][tr][q], sems[s]).start()

    def wait_set(s):
        for q in range(4):
            for tr in range(4):
                pltpu.make_async_copy(
                    ft_hbm.at[pl.ds(0, 8), pl.ds(0, 128)],
                    fb[s][tr][q], sems[s]).wait()
                pltpu.make_async_copy(
                    pt_hbm.at[pl.ds(0, 8), pl.ds(0, 128)],
                    pb[s][tr][q], sems[s]).wait()

    def extract(g, s):
        gi = jnp.full((16,), 0, jnp.int32) + g + jnp.minimum(iota16, 1)
        ov = plsc.load_gather(offs_v, [gi])
        os0 = ov[0]
        os1 = ov[1]
        nb = lax.shift_right_logical(os1 - os0 + 15, 4)

        def one_batch(b, z):
            for h in range(2):
                pltpu.make_async_copy(
                    stg[z][h], img_hbm.at[jbufs[z][h]], ssems[z]).wait()
            idx = os0 + b * 16 + iota16
            valid = idx < os1
            idxc = jnp.minimum(idx, jnp.maximum(os1 - 1, 0))
            kvec = plsc.load_gather(hkv, [idxc])
            tcq = lax.shift_right_logical(kvec, 21) - g * 4
            wcv = kvec & 127
            dmy = NUM_ENT + (((wid * NG + g + b) * 16 + iota16)
                             & (IMG_PAD - 1))
            jv = jnp.where(valid,
                           lax.shift_right_logical(kvec, 7) & 16383,
                           dmy)
            plsc.store_scatter(jbufs[z][0], [iota16], jv, mask=lane_lo)
            plsc.store_scatter(jbufs[z][1], [iota16 - 8], jv, mask=lane_hi)

            for d in range(EMBED_DIM):
                tr = d // 8
                rowv = jnp.full((16,), 0, jnp.int32) + (d % 8)
                gf = None
                gp = None
                for q in range(4):
                    fq = plsc.load_gather(fb[s][tr][q], [rowv, wcv])
                    pq = plsc.load_gather(pb[s][tr][q], [rowv, wcv])
                    if gf is None:
                        gf, gp = fq, pq
                    else:
                        sel = tcq == q
                        gf = jnp.where(sel, fq, gf)
                        gp = jnp.where(sel, pq, gp)
                sv = gf + gp
                dv = jnp.full((16,), 0, jnp.int32) + d
                plsc.store_scatter(stg[z][0], [iota16, dv], sv, mask=lane_lo)
                plsc.store_scatter(stg[z][1], [iota16 - 8, dv], sv,
                                   mask=lane_hi)

            for h in range(2):
                pltpu.make_async_copy(
                    stg[z][h], img_hbm.at[jbufs[z][h]], ssems[z]).start()

        def batch_pair(b2, carry2):
            one_batch(b2 * 2, 0)
            one_batch(b2 * 2 + 1, 1)
            return carry2

        lax.fori_loop(0, lax.shift_right_logical(nb + 1, 1), batch_pair, 0)

    dmy0 = NUM_ENT + ((wid * 64 + iota16) & (IMG_PAD - 1))
    for z in range(2):
        plsc.store_scatter(jbufs[z][0], [iota16], dmy0 + z * 16,
                           mask=lane_lo)
        plsc.store_scatter(jbufs[z][1], [iota16 - 8], dmy0 + z * 16 + 8,
                           mask=lane_hi)
        for h in range(2):
            pltpu.make_async_copy(
                stg[z][h], img_hbm.at[jbufs[z][h]], ssems[z]).start()

    fire_set(jnp.int32(0), 0)

    def pair_body(g2, carry):
        g0 = g2 * 2
        fire_set(g0 + 1, 1)
        wait_set(0)
        extract(g0, 0)
        fire_set(g0 + 2, 0)
        wait_set(1)
        extract(g0 + 1, 1)
        return carry

    lax.fori_loop(0, NG // 2, pair_body, 0)
    wait_set(0)
    for z in range(2):
        for h in range(2):
            pltpu.make_async_copy(
                stg[z][h], img_hbm.at[jbufs[z][h]], ssems[z]).wait()


@functools.partial(
    pl.kernel,
    mesh=_mesh,
    out_type=jax.ShapeDtypeStruct((EMBED_DIM, NUM_ENT), jnp.float32),
    scratch_types=[
        pltpu.VMEM((B_PER_W, EMBED_DIM), jnp.float32),
        pltpu.VMEM((EMBED_DIM, B_PER_W), jnp.float32),
    ],
    compiler_params=pltpu.CompilerParams(
        use_tc_tiling_on_sc=False, needs_layout_passes=False),
)
def _k2(img_hbm, out_hbm, buf, out_t):
    wid, _, _ = _wid_lo()
    base = wid * B_PER_W
    pltpu.sync_copy(
        img_hbm.at[pl.ds(base, B_PER_W), pl.ds(0, EMBED_DIM)], buf)
    iota16 = lax.iota(jnp.int32, 16)

    def body(i, carry):
        d = i & 31
        e16 = i >> 5
        rowv = iota16 + e16 * 16
        colv = jnp.full((16,), 0, jnp.int32) + d
        out_t[d, pl.ds(e16 * 16, 16)] = plsc.load_gather(buf, [rowv, colv])
        return carry

    lax.fori_loop(0, EMBED_DIM * (B_PER_W // 16), body, 0, unroll=4)
    pltpu.sync_copy(out_t, out_hbm.at[:, pl.ds(base, B_PER_W)])


def kernel(nodes, feat_table, pos_table):
    n = nodes.astype(jnp.int32)
    hko, offs = _k0(n)
    img = _k1(hko, offs, feat_table.T, pos_table.T)
    return _k2(img)


# speedup vs baseline: 1.9628x; 1.2577x over previous
import functools

import jax
import jax.numpy as jnp
from jax import lax
from jax.experimental import pallas as pl
from jax.experimental.pallas import tpu as pltpu
from jax.experimental.pallas import tpu_sc as plsc

NUM_NODES = 1000000
EMBED_DIM = 32
NUM_ENT = 16384

NC, NS = 2, 16
NW = NC * NS
B_PER_W = NUM_ENT // NW
N_TC = 7813
TC_BASE = N_TC // NW
TC_REM = N_TC - TC_BASE * NW
NG = 62
HCAP = 1152
OCAP = 128
IMG_PAD = 2048
IMG_H = NUM_ENT + IMG_PAD

_mesh = plsc.VectorSubcoreMesh(core_axis_name="c", subcore_axis_name="s")


def _wid_lo():
    wid = lax.axis_index("s") * NC + lax.axis_index("c")
    lo = wid * TC_BASE + jnp.minimum(wid, TC_REM)
    hi = lo + TC_BASE + jnp.where(wid < TC_REM, 1, 0)
    return wid, lo, hi


@functools.partial(
    pl.kernel,
    mesh=_mesh,
    out_type=(
        jax.ShapeDtypeStruct((NW * HCAP,), jnp.int32),
        jax.ShapeDtypeStruct((NW * OCAP,), jnp.int32),
    ),
    scratch_types=[
        pltpu.VMEM((NUM_ENT,), jnp.int32),
        pltpu.VMEM((HCAP,), jnp.int32),
        pltpu.VMEM((HCAP,), jnp.int32),
        pltpu.VMEM((OCAP,), jnp.int32),
    ],
    compiler_params=pltpu.CompilerParams(
        use_tc_tiling_on_sc=False, needs_layout_passes=False),
)
def _k0(nodes_hbm, hko_hbm, offs_hbm, nv, hk1, hk2, offs_v):
    wid, lo, hi = _wid_lo()
    pltpu.sync_copy(nodes_hbm, nv)
    iota16 = lax.iota(jnp.int32, 16)

    def scan_body(i, off):
        c = nv[pl.ds(i * 16, 16)]
        tc = lax.shift_right_logical(c, 7)
        m = (tc >= lo) & (tc < hi)
        j = iota16 + i * 16
        key = (lax.shift_left(tc - lo, 21)
               | lax.shift_left(j, 7) | (c & 127))
        offc = jnp.minimum(off, HCAP - 16)
        plsc.store_compressed(hk1.at[pl.ds(offc, 16)], key, mask=m)
        return off + jnp.sum(jnp.where(m, 1, 0))

    nh = lax.fori_loop(0, NUM_ENT // 16, scan_body, jnp.int32(0), unroll=4)
    nsteps = lax.shift_right_logical(nh + 15, 4)

    def group_body(g, off2):
        gv = jnp.full((16,), 0, jnp.int32) + g
        ov = jnp.full((16,), 0, jnp.int32) + off2
        plsc.store_scatter(offs_v, [gv], ov, mask=iota16 < 1)

        def inner(i, o2):
            pos = i * 16 + iota16
            key = hk1[pl.ds(i * 16, 16)]
            tcl = lax.shift_right_logical(key, 21)
            m = (pos < nh) & (lax.shift_right_logical(tcl, 2) == g)
            oc = jnp.minimum(o2, HCAP - 16)
            plsc.store_compressed(hk2.at[pl.ds(oc, 16)], key, mask=m)
            return o2 + jnp.sum(jnp.where(m, 1, 0))

        return lax.fori_loop(0, nsteps, inner, off2)

    off_end = lax.fori_loop(0, NG, group_body, jnp.int32(0))
    gv = jnp.full((16,), 0, jnp.int32) + NG
    ov = jnp.full((16,), 0, jnp.int32) + off_end
    plsc.store_scatter(offs_v, [gv], ov, mask=iota16 < 1)

    pltpu.sync_copy(hk2, hko_hbm.at[pl.ds(wid * HCAP, HCAP)])
    pltpu.sync_copy(offs_v, offs_hbm.at[pl.ds(wid * OCAP, OCAP)])


@functools.partial(
    pl.kernel,
    mesh=_mesh,
    out_type=jax.ShapeDtypeStruct((IMG_H, 128), jnp.float32),
    scratch_types=[
        pltpu.VMEM((HCAP,), jnp.int32),
        pltpu.VMEM((OCAP,), jnp.int32),
        [[[pltpu.VMEM((8, 128), jnp.float32) for _ in range(4)]
          for _ in range(4)] for _ in range(2)],
        [[[pltpu.VMEM((8, 128), jnp.float32) for _ in range(4)]
          for _ in range(4)] for _ in range(2)],
        [[pltpu.VMEM((8, 128), jnp.float32) for _ in range(2)]
         for _ in range(2)],
        [[pltpu.VMEM((8,), jnp.int32) for _ in range(2)] for _ in range(2)],
        [pltpu.SemaphoreType.DMA for _ in range(2)],
        [pltpu.SemaphoreType.DMA for _ in range(2)],
    ],
    compiler_params=pltpu.CompilerParams(
        needs_layout_passes=False, disable_bounds_checks=True),
)
def _k1(hko_hbm, offs_hbm, ft_hbm, pt_hbm, img_hbm,
        hkv, offs_v, fb, pb, stg, jbufs, sems, ssems):
    wid, lo, hi = _wid_lo()
    pltpu.sync_copy(hko_hbm.at[pl.ds(wid * HCAP, HCAP)], hkv)
    pltpu.sync_copy(offs_hbm.at[pl.ds(wid * OCAP, OCAP)], offs_v)
    iota16 = lax.iota(jnp.int32, 16)
    lane_lo = iota16 < 8
    lane_hi = iota16 >= 8

    def fire_set(g, s):
        for q in range(4):
            tc = jnp.minimum(lo + g * 4 + q, N_TC - 1)
            cstart = pl.multiple_of(tc * 128, 128)
            for tr in range(4):
                pltpu.make_async_copy(
                    ft_hbm.at[pl.ds(tr * 8, 8), pl.ds(cstart, 128)],
                    fb[s][tr][q], sems[s]).start()
                pltpu.make_async_copy(
                    pt_hbm.at[pl.ds(tr * 8, 8), pl.ds(cstart, 128)],
                    pb[s][tr][q], sems[s]).start()

    def wait_set(s):
        for q in range(4):
            for tr in range(4):
                pltpu.make_async_copy(
                    ft_hbm.at[pl.ds(0, 8), pl.ds(0, 128)],
                    fb[s][tr][q], sems[s]).wait()
                pltpu.make_async_copy(
                    pt_hbm.at[pl.ds(0, 8), pl.ds(0, 128)],
                    pb[s][tr][q], sems[s]).wait()

    def extract(g, s):
        gi = jnp.full((16,), 0, jnp.int32) + g + jnp.minimum(iota16, 1)
        ov = plsc.load_gather(offs_v, [gi])
        os0 = ov[0]
        os1 = ov[1]
        nb = lax.shift_right_logical(os1 - os0 + 15, 4)

        def one_batch(b, z):
            @pl.when(b < nb)
            def _batch():
                _one_batch_body(b, z)

        def _one_batch_body(b, z):
            for h in range(2):
                pltpu.make_async_copy(
                    stg[z][h], img_hbm.at[jbufs[z][h]], ssems[z]).wait()
            idx = os0 + b * 16 + iota16
            valid = idx < os1
            idxc = jnp.minimum(idx, jnp.maximum(os1 - 1, 0))
            kvec = plsc.load_gather(hkv, [idxc])
            tcq = lax.shift_right_logical(kvec, 21) - g * 4
            wcv = kvec & 127
            dmy = NUM_ENT + (((wid * NG + g + b) * 16 + iota16)
                             & (IMG_PAD - 1))
            jv = jnp.where(valid,
                           lax.shift_right_logical(kvec, 7) & 16383,
                           dmy)
            plsc.store_scatter(jbufs[z][0], [iota16], jv, mask=lane_lo)
            plsc.store_scatter(jbufs[z][1], [iota16 - 8], jv, mask=lane_hi)

            for d in range(EMBED_DIM):
                tr = d // 8
                rowv = jnp.full((16,), 0, jnp.int32) + (d % 8)
                gf = None
                gp = None
                for q in range(4):
                    fq = plsc.load_gather(fb[s][tr][q], [rowv, wcv])
                    pq = plsc.load_gather(pb[s][tr][q], [rowv, wcv])
                    if gf is None:
                        gf, gp = fq, pq
                    else:
                        sel = tcq == q
                        gf = jnp.where(sel, fq, gf)
                        gp = jnp.where(sel, pq, gp)
                sv = gf + gp
                dv = jnp.full((16,), 0, jnp.int32) + d
                plsc.store_scatter(stg[z][0], [iota16, dv], sv, mask=lane_lo)
                plsc.store_scatter(stg[z][1], [iota16 - 8, dv], sv,
                                   mask=lane_hi)

            for h in range(2):
                pltpu.make_async_copy(
                    stg[z][h], img_hbm.at[jbufs[z][h]], ssems[z]).start()

        def batch_pair(b2, carry2):
            one_batch(b2 * 2, 0)
            one_batch(b2 * 2 + 1, 1)
            return carry2

        lax.fori_loop(0, lax.shift_right_logical(nb + 1, 1), batch_pair, 0)

    dmy0 = NUM_ENT + ((wid * 64 + iota16) & (IMG_PAD - 1))
    for z in range(2):
        plsc.store_scatter(jbufs[z][0], [iota16], dmy0 + z * 16,
                           mask=lane_lo)
        plsc.store_scatter(jbufs[z][1], [iota16 - 8], dmy0 + z * 16 + 8,
                           mask=lane_hi)
        for h in range(2):
            pltpu.make_async_copy(
                stg[z][h], img_hbm.at[jbufs[z][h]], ssems[z]).start()

    fire_set(jnp.int32(0), 0)

    def pair_body(g2, carry):
        g0 = g2 * 2
        fire_set(g0 + 1, 1)
        wait_set(0)
        extract(g0, 0)
        fire_set(g0 + 2, 0)
        wait_set(1)
        extract(g0 + 1, 1)
        return carry

    lax.fori_loop(0, NG // 2, pair_body, 0)
    wait_set(0)
    for z in range(2):
        for h in range(2):
            pltpu.make_async_copy(
                stg[z][h], img_hbm.at[jbufs[z][h]], ssems[z]).wait()


@functools.partial(
    pl.kernel,
    mesh=_mesh,
    out_type=jax.ShapeDtypeStruct((EMBED_DIM, NUM_ENT), jnp.float32),
    scratch_types=[
        pltpu.VMEM((B_PER_W, EMBED_DIM), jnp.float32),
        pltpu.VMEM((EMBED_DIM, B_PER_W), jnp.float32),
    ],
    compiler_params=pltpu.CompilerParams(
        use_tc_tiling_on_sc=False, needs_layout_passes=False),
)
def _k2(img_hbm, out_hbm, buf, out_t):
    wid, _, _ = _wid_lo()
    base = wid * B_PER_W
    pltpu.sync_copy(
        img_hbm.at[pl.ds(base, B_PER_W), pl.ds(0, EMBED_DIM)], buf)
    iota16 = lax.iota(jnp.int32, 16)

    def body(i, carry):
        d = i & 31
        e16 = i >> 5
        rowv = iota16 + e16 * 16
        colv = jnp.full((16,), 0, jnp.int32) + d
        out_t[d, pl.ds(e16 * 16, 16)] = plsc.load_gather(buf, [rowv, colv])
        return carry

    lax.fori_loop(0, EMBED_DIM * (B_PER_W // 16), body, 0, unroll=4)
    pltpu.sync_copy(out_t, out_hbm.at[:, pl.ds(base, B_PER_W)])


def kernel(nodes, feat_table, pos_table):
    n = nodes.astype(jnp.int32)
    hko, offs = _k0(n)
    img = _k1(hko, offs, feat_table.T, pos_table.T)
    return _k2(img)
